# baseline (speedup 1.0000x reference)
import functools

import jax
import jax.numpy as jnp
from jax import lax
from jax.experimental import pallas as pl
from jax.experimental.pallas import tpu as pltpu
from jax.experimental.pallas import tpu_sc as plsc

N = 10000
E = 320000
D = 128

NC = 2
NS = 16
NW = NC * NS

C = 128
EW = E // NW
NR = 80
PAD = NR * C - EW
HALF = NR // 2

NA = N + 8

RPB = 80
NB = N // RPB
KMAX = -(-NB // NS)


@functools.partial(
    pl.kernel,
    out_type=jax.ShapeDtypeStruct((NC, N, D), jnp.float32),
    mesh=plsc.VectorSubcoreMesh(
        core_axis_name="c", subcore_axis_name="s",
        num_cores=NC, num_subcores=NS),
    scratch_types=[
        pltpu.VMEM((HALF, C), jnp.int32),
        pltpu.VMEM((HALF, C), jnp.int32),
        pltpu.VMEM((C, D), jnp.float32),
        pltpu.VMEM((C, D), jnp.float32),
        pltpu.VMEM_SHARED((NA, D), jnp.float32),
        pltpu.SemaphoreType.DMA,
        pltpu.SemaphoreType.DMA,
    ],
)
def _sc_aggregate(x_hbm, src_hbm, dst_hbm, out_hbm,
                  sidx, didx, rows, rows_b, agg, sem, sem_b):
    cid = lax.axis_index("c")
    sid = lax.axis_index("s")
    wid = cid * NS + sid

    def zrow(i, _):
        def zlane(j, _):
            rows[i, pl.ds(j * 16, 16)] = jnp.zeros((16,), jnp.float32)
            return 0
        return lax.fori_loop(0, D // 16, zlane, 0)
    lax.fori_loop(0, C, zrow, 0)

    def zcp(k, _):
        cb = sid + k * NS
        @pl.when(cb < NB)
        def _():
            pltpu.sync_copy(rows.at[pl.ds(0, RPB)],
                            agg.at[pl.ds(cb * RPB, RPB)])
        return 0
    lax.fori_loop(0, KMAX, zcp, 0)
    plsc.subcore_barrier()

    def half(h, _):
        r0 = h * HALF
        pltpu.sync_copy(src_hbm.at[wid, pl.ds(r0, HALF)], sidx)
        pltpu.sync_copy(dst_hbm.at[wid, pl.ds(r0, HALF)], didx)
        def chunk(j, _):
            pltpu.async_copy(x_hbm.at[sidx.at[j]], rows, sem).wait()
            pltpu.sync_copy(rows, agg.at[didx.at[j]], add=True)
            return 0
        lax.fori_loop(0, HALF, chunk, 0)
        return 0
    lax.fori_loop(0, NR // HALF, half, 0)
    plsc.subcore_barrier()

    def ocp(k, _):
        cb = sid + k * NS
        @pl.when(cb < NB)
        def _():
            r0 = cb * RPB
            pltpu.sync_copy(agg.at[pl.ds(r0, RPB)], rows.at[pl.ds(0, RPB)])
            pltpu.sync_copy(rows.at[pl.ds(0, RPB)],
                            out_hbm.at[cid, pl.ds(r0, RPB)])
        return 0
    lax.fori_loop(0, KMAX, ocp, 0)


BR = 1000


def _tc_update_body(x_ref, p0_ref, p1_ref, w_ref, b_ref, o_ref):
    h = x_ref[...] + p0_ref[...] + p1_ref[...]
    acc = jnp.dot(h, w_ref[...], preferred_element_type=jnp.float32)
    o_ref[...] = jnp.maximum(acc + b_ref[...], 0.0)


def _tc_update(x, p0, p1, W, b2):
    return pl.pallas_call(
        _tc_update_body,
        grid=(N // BR,),
        in_specs=[
            pl.BlockSpec((BR, D), lambda i: (i, 0)),
            pl.BlockSpec((BR, D), lambda i: (i, 0)),
            pl.BlockSpec((BR, D), lambda i: (i, 0)),
            pl.BlockSpec((D, D), lambda i: (0, 0)),
            pl.BlockSpec((1, D), lambda i: (0, 0)),
        ],
        out_specs=pl.BlockSpec((BR, D), lambda i: (i, 0)),
        out_shape=jax.ShapeDtypeStruct((N, D), jnp.float32),
    )(x, p0, p1, W, b2)


def kernel(x, edge_index, W, b):
    src = edge_index[0].astype(jnp.int32).reshape(NW, EW)
    dst = edge_index[1].astype(jnp.int32).reshape(NW, EW)
    srcp = jnp.pad(src, ((0, 0), (0, PAD))).reshape(NW, NR, C)
    dstp = jnp.pad(dst, ((0, 0), (0, PAD)), constant_values=N)
    dstp = dstp.reshape(NW, NR, C)
    parts = _sc_aggregate(x, srcp, dstp)
    return _tc_update(x, parts[0], parts[1], W, b.reshape(1, D))


# speedup vs baseline: 1.1446x; 1.1446x over previous
import functools

import jax
import jax.numpy as jnp
from jax import lax
from jax.experimental import pallas as pl
from jax.experimental.pallas import tpu as pltpu
from jax.experimental.pallas import tpu_sc as plsc

N = 10000
E = 320000
D = 128

NC = 2
NS = 16
NW = NC * NS

C = 128
EW = E // NW
NR = 80
PAD = NR * C - EW
HALF = NR // 2

NA = N + 128

RPB = 80
NB = N // RPB
KMAX = -(-NB // NS)


@functools.partial(
    pl.kernel,
    out_type=jax.ShapeDtypeStruct((NC, N, D), jnp.float32),
    mesh=plsc.VectorSubcoreMesh(
        core_axis_name="c", subcore_axis_name="s",
        num_cores=NC, num_subcores=NS),
    scratch_types=[
        pltpu.VMEM((HALF, C), jnp.int32),
        pltpu.VMEM((HALF, C), jnp.int32),
        pltpu.VMEM((C, D), jnp.float32),
        pltpu.VMEM((C, D), jnp.float32),
        pltpu.VMEM_SHARED((NA, D), jnp.float32),
        pltpu.SemaphoreType.DMA,
        pltpu.SemaphoreType.DMA,
    ],
)
def _sc_aggregate(x_hbm, src_hbm, dst_hbm, out_hbm,
                  sidx, didx, rows, rows_b, agg, sem, sem_b):
    cid = lax.axis_index("c")
    sid = lax.axis_index("s")
    wid = cid * NS + sid

    def zrow(i, _):
        def zlane(j, _):
            rows[i, pl.ds(j * 16, 16)] = jnp.zeros((16,), jnp.float32)
            return 0
        return lax.fori_loop(0, D // 16, zlane, 0)
    lax.fori_loop(0, C, zrow, 0)

    def zcp(k, _):
        cb = sid + k * NS
        @pl.when(cb < NB)
        def _():
            pltpu.sync_copy(rows.at[pl.ds(0, RPB)],
                            agg.at[pl.ds(cb * RPB, RPB)])
        return 0
    lax.fori_loop(0, KMAX, zcp, 0)
    plsc.subcore_barrier()

    def half(h, _):
        r0 = h * HALF
        pltpu.sync_copy(src_hbm.at[wid, pl.ds(r0, HALF)], sidx)
        pltpu.sync_copy(dst_hbm.at[wid, pl.ds(r0, HALF)], didx)
        pltpu.async_copy(x_hbm.at[sidx.at[0]], rows, sem)

        def chunk_pair(i, _):
            j = 2 * i
            pltpu.async_copy(x_hbm.at[sidx.at[j + 1]], rows_b, sem_b)
            pltpu.make_async_copy(x_hbm.at[sidx.at[j]], rows, sem).wait()
            pltpu.sync_copy(rows, agg.at[didx.at[j]], add=True)

            @pl.when(j + 2 < HALF)
            def _():
                pltpu.async_copy(x_hbm.at[sidx.at[j + 2]], rows, sem)
            pltpu.make_async_copy(
                x_hbm.at[sidx.at[j + 1]], rows_b, sem_b).wait()
            pltpu.sync_copy(rows_b, agg.at[didx.at[j + 1]], add=True)
            return 0
        lax.fori_loop(0, HALF // 2, chunk_pair, 0)
        return 0
    lax.fori_loop(0, NR // HALF, half, 0)
    plsc.subcore_barrier()

    def ocp(k, _):
        cb = sid + k * NS
        @pl.when(cb < NB)
        def _():
            r0 = cb * RPB
            pltpu.sync_copy(agg.at[pl.ds(r0, RPB)], rows.at[pl.ds(0, RPB)])
            pltpu.sync_copy(rows.at[pl.ds(0, RPB)],
                            out_hbm.at[cid, pl.ds(r0, RPB)])
        return 0
    lax.fori_loop(0, KMAX, ocp, 0)


BR = 1000


def _tc_update_body(x_ref, p0_ref, p1_ref, w_ref, b_ref, o_ref):
    h = x_ref[...] + p0_ref[...] + p1_ref[...]
    acc = jnp.dot(h, w_ref[...], preferred_element_type=jnp.float32)
    o_ref[...] = jnp.maximum(acc + b_ref[...], 0.0)


def _tc_update(x, p0, p1, W, b2):
    return pl.pallas_call(
        _tc_update_body,
        grid=(N // BR,),
        in_specs=[
            pl.BlockSpec((BR, D), lambda i: (i, 0)),
            pl.BlockSpec((BR, D), lambda i: (i, 0)),
            pl.BlockSpec((BR, D), lambda i: (i, 0)),
            pl.BlockSpec((D, D), lambda i: (0, 0)),
            pl.BlockSpec((1, D), lambda i: (0, 0)),
        ],
        out_specs=pl.BlockSpec((BR, D), lambda i: (i, 0)),
        out_shape=jax.ShapeDtypeStruct((N, D), jnp.float32),
    )(x, p0, p1, W, b2)


def kernel(x, edge_index, W, b):
    src = edge_index[0].astype(jnp.int32).reshape(NW, EW)
    dst = edge_index[1].astype(jnp.int32).reshape(NW, EW)
    srcp = jnp.pad(src, ((0, 0), (0, PAD))).reshape(NW, NR, C)
    dpad = N + (jnp.arange(PAD, dtype=jnp.int32) % (NA - N))
    dpad = jnp.broadcast_to(dpad, (NW, PAD))
    dstp = jnp.concatenate([dst, dpad], axis=1).reshape(NW, NR, C)
    parts = _sc_aggregate(x, srcp, dstp)
    return _tc_update(x, parts[0], parts[1], W, b.reshape(1, D))


# speedup vs baseline: 3.0119x; 2.6313x over previous
import functools

import jax
import jax.numpy as jnp
from jax import lax
from jax.experimental import pallas as pl
from jax.experimental.pallas import tpu as pltpu
from jax.experimental.pallas import tpu_sc as plsc

N = 10000
E = 320000
D = 128

NC = 2
NS = 16
NW = NC * NS

C = 80
EW = E // NW
NCH = EW // C

RPB = C
NB = N // RPB
KMAX = -(-NB // NS)


@functools.partial(
    pl.kernel,
    out_type=jax.ShapeDtypeStruct((NC, N, D), jnp.float32),
    mesh=plsc.VectorSubcoreMesh(
        core_axis_name="c", subcore_axis_name="s",
        num_cores=NC, num_subcores=NS),
    scratch_types=[
        pltpu.VMEM((EW,), jnp.int32),
        pltpu.VMEM((NCH, C), jnp.int32),
        pltpu.VMEM((C, D), jnp.float32),
        pltpu.VMEM((C, D), jnp.float32),
        pltpu.VMEM_SHARED((N, D), jnp.float32),
        pltpu.SemaphoreType.DMA,
        pltpu.SemaphoreType.DMA,
    ],
)
def _sc_aggregate(x_hbm, srcf_hbm, dst_hbm, out_hbm,
                  sall, dall, rows, rows_b, agg, sem, sem_b):
    cid = lax.axis_index("c")
    sid = lax.axis_index("s")
    wid = cid * NS + sid

    pltpu.sync_copy(srcf_hbm.at[pl.ds(wid * EW, EW)], sall)
    pltpu.sync_copy(dst_hbm.at[wid], dall)

    def zrow(i, _):
        def zlane(j, _):
            rows[i, pl.ds(j * 16, 16)] = jnp.zeros((16,), jnp.float32)
            return 0
        return lax.fori_loop(0, D // 16, zlane, 0)
    lax.fori_loop(0, C, zrow, 0)

    def zcp(k, _):
        cb = sid + k * NS
        @pl.when(cb < NB)
        def _():
            pltpu.sync_copy(rows, agg.at[pl.ds(cb * RPB, RPB)])
        return 0
    lax.fori_loop(0, KMAX, zcp, 0)
    plsc.subcore_barrier()

    pltpu.async_copy(x_hbm.at[sall.at[pl.ds(0, C)]], rows, sem)

    def chunk_pair(i, _):
        j = 2 * i
        pltpu.async_copy(
            x_hbm.at[sall.at[pl.ds((j + 1) * C, C)]], rows_b, sem_b)
        pltpu.make_async_copy(
            x_hbm.at[sall.at[pl.ds(j * C, C)]], rows, sem).wait()
        pltpu.sync_copy(rows, agg.at[dall.at[j]], add=True)

        @pl.when(j + 2 < NCH)
        def _():
            pltpu.async_copy(
                x_hbm.at[sall.at[pl.ds((j + 2) * C, C)]], rows, sem)
        pltpu.make_async_copy(
            x_hbm.at[sall.at[pl.ds((j + 1) * C, C)]], rows_b, sem_b).wait()
        pltpu.sync_copy(rows_b, agg.at[dall.at[j + 1]], add=True)
        return 0
    lax.fori_loop(0, NCH // 2, chunk_pair, 0)

    pltpu.make_async_copy(
        x_hbm.at[sall.at[pl.ds((NCH - 1) * C, C)]], rows, sem).wait()
    pltpu.sync_copy(rows, agg.at[dall.at[NCH - 1]], add=True)
    plsc.subcore_barrier()

    def ocp(k, _):
        cb = sid + k * NS
        @pl.when(cb < NB)
        def _():
            r0 = cb * RPB
            pltpu.sync_copy(agg.at[pl.ds(r0, RPB)], rows)
            pltpu.sync_copy(rows, out_hbm.at[cid, pl.ds(r0, RPB)])
        return 0
    lax.fori_loop(0, KMAX, ocp, 0)


BR = 1000


def _tc_update_body(x_ref, p0_ref, p1_ref, w_ref, b_ref, o_ref):
    h = x_ref[...] + p0_ref[...] + p1_ref[...]
    acc = jnp.dot(h, w_ref[...], preferred_element_type=jnp.float32)
    o_ref[...] = jnp.maximum(acc + b_ref[...], 0.0)


def _tc_update(x, p0, p1, W, b2):
    return pl.pallas_call(
        _tc_update_body,
        grid=(N // BR,),
        in_specs=[
            pl.BlockSpec((BR, D), lambda i: (i, 0)),
            pl.BlockSpec((BR, D), lambda i: (i, 0)),
            pl.BlockSpec((BR, D), lambda i: (i, 0)),
            pl.BlockSpec((D, D), lambda i: (0, 0)),
            pl.BlockSpec((1, D), lambda i: (0, 0)),
        ],
        out_specs=pl.BlockSpec((BR, D), lambda i: (i, 0)),
        out_shape=jax.ShapeDtypeStruct((N, D), jnp.float32),
    )(x, p0, p1, W, b2)


def kernel(x, edge_index, W, b):
    src = edge_index[0].astype(jnp.int32)
    dst = edge_index[1].astype(jnp.int32).reshape(NW, NCH, C)
    parts = _sc_aggregate(x, src, dst)
    return _tc_update(x, parts[0], parts[1], W, b.reshape(1, D))


# speedup vs baseline: 3.0273x; 1.0051x over previous
import functools

import jax
import jax.numpy as jnp
from jax import lax
from jax.experimental import pallas as pl
from jax.experimental.pallas import tpu as pltpu
from jax.experimental.pallas import tpu_sc as plsc

N = 10000
E = 320000
D = 128

NC = 2
NS = 16
NW = NC * NS

C = 80
EW = E // NW
NCH = EW // C

RPB = C
NB = N // RPB
KMAX = -(-NB // NS)


@functools.partial(
    pl.kernel,
    out_type=jax.ShapeDtypeStruct((NC, N, D), jnp.float32),
    mesh=plsc.VectorSubcoreMesh(
        core_axis_name="c", subcore_axis_name="s",
        num_cores=NC, num_subcores=NS),
    scratch_types=[
        pltpu.VMEM((EW,), jnp.int32),
        pltpu.VMEM((NCH, C), jnp.int32),
        pltpu.VMEM((C, D), jnp.float32),
        pltpu.VMEM((C, D), jnp.float32),
        pltpu.VMEM_SHARED((N, D), jnp.float32),
        pltpu.SemaphoreType.DMA,
        pltpu.SemaphoreType.DMA,
    ],
)
def _sc_aggregate(x_hbm, srcf_hbm, dst_hbm, out_hbm,
                  sall, dall, rows, rows_b, agg, sem, sem_b):
    cid = lax.axis_index("c")
    sid = lax.axis_index("s")
    wid = cid * NS + sid

    pltpu.sync_copy(srcf_hbm.at[pl.ds(wid * EW, EW)], sall)
    pltpu.sync_copy(dst_hbm.at[wid], dall)

    def zrow(i, _):
        def zlane(j, _):
            rows[i, pl.ds(j * 16, 16)] = jnp.zeros((16,), jnp.float32)
            return 0
        return lax.fori_loop(0, D // 16, zlane, 0)
    lax.fori_loop(0, C, zrow, 0)

    def zcp(k, _):
        cb = sid + k * NS
        @pl.when(cb < NB)
        def _():
            pltpu.sync_copy(rows, agg.at[pl.ds(cb * RPB, RPB)])
        return 0
    lax.fori_loop(0, KMAX, zcp, 0)
    plsc.subcore_barrier()

    pltpu.async_copy(x_hbm.at[sall.at[pl.ds(0, C)]], rows, sem)

    def chunk_pair(i, _):
        j = 2 * i
        pltpu.async_copy(
            x_hbm.at[sall.at[pl.ds((j + 1) * C, C)]], rows_b, sem_b)
        pltpu.make_async_copy(
            x_hbm.at[sall.at[pl.ds(j * C, C)]], rows, sem).wait()
        pltpu.sync_copy(rows, agg.at[dall.at[j]], add=True)

        @pl.when(j + 2 < NCH)
        def _():
            pltpu.async_copy(
                x_hbm.at[sall.at[pl.ds((j + 2) * C, C)]], rows, sem)
        pltpu.make_async_copy(
            x_hbm.at[sall.at[pl.ds((j + 1) * C, C)]], rows_b, sem_b).wait()
        pltpu.sync_copy(rows_b, agg.at[dall.at[j + 1]], add=True)
        return 0
    lax.fori_loop(0, NCH // 2, chunk_pair, 0)

    pltpu.make_async_copy(
        x_hbm.at[sall.at[pl.ds((NCH - 1) * C, C)]], rows, sem).wait()
    pltpu.sync_copy(rows, agg.at[dall.at[NCH - 1]], add=True)
    plsc.subcore_barrier()

    def ocp(k, _):
        cb = sid + k * NS
        @pl.when(cb < NB)
        def _():
            r0 = cb * RPB
            pltpu.sync_copy(agg.at[pl.ds(r0, RPB)], rows)
            pltpu.sync_copy(rows, out_hbm.at[cid, pl.ds(r0, RPB)])
        return 0
    lax.fori_loop(0, KMAX, ocp, 0)


BR = 2000


def _tc_xw_body(x_ref, w_ref, b_ref, o_ref):
    acc = jnp.dot(x_ref[...], w_ref[...], preferred_element_type=jnp.float32)
    o_ref[...] = acc + b_ref[...]


def _tc_xw(x, W, b2):
    return pl.pallas_call(
        _tc_xw_body,
        grid=(N // BR,),
        in_specs=[
            pl.BlockSpec((BR, D), lambda i: (i, 0)),
            pl.BlockSpec((D, D), lambda i: (0, 0)),
            pl.BlockSpec((1, D), lambda i: (0, 0)),
        ],
        out_specs=pl.BlockSpec((BR, D), lambda i: (i, 0)),
        out_shape=jax.ShapeDtypeStruct((N, D), jnp.float32),
    )(x, W, b2)


def _tc_fin_body(y_ref, p0_ref, p1_ref, w_ref, o_ref):
    h = p0_ref[...] + p1_ref[...]
    acc = jnp.dot(h, w_ref[...], preferred_element_type=jnp.float32)
    o_ref[...] = jnp.maximum(y_ref[...] + acc, 0.0)


def _tc_fin(y, p0, p1, W):
    return pl.pallas_call(
        _tc_fin_body,
        grid=(N // BR,),
        in_specs=[
            pl.BlockSpec((BR, D), lambda i: (i, 0)),
            pl.BlockSpec((BR, D), lambda i: (i, 0)),
            pl.BlockSpec((BR, D), lambda i: (i, 0)),
            pl.BlockSpec((D, D), lambda i: (0, 0)),
        ],
        out_specs=pl.BlockSpec((BR, D), lambda i: (i, 0)),
        out_shape=jax.ShapeDtypeStruct((N, D), jnp.float32),
    )(y, p0, p1, W)


def kernel(x, edge_index, W, b):
    src = edge_index[0].astype(jnp.int32)
    dst = edge_index[1].astype(jnp.int32).reshape(NW, NCH, C)
    parts = _sc_aggregate(x, src, dst)
    y = _tc_xw(x, W, b.reshape(1, D))
    return _tc_fin(y, parts[0], parts[1], W)
